# baseline (device time: 47472 ns/iter reference)
import os

import jax
import jax.numpy as jnp
from jax import lax
from jax.experimental import pallas as pl
from jax.experimental.pallas import tpu as pltpu

N_DEV = 4
N_HOP = N_DEV - 1
SUB = int(os.environ.get("KSUB", "4"))
_MODE = os.environ.get("KMODE", "full")
_WIRE = os.environ.get("KWIRE", "bf16")


def kernel(x, w_mat):
    m_per, k = x.shape
    _, n_per = w_mat.shape
    m_half = m_per // 2
    m_sub = m_half // SUB
    wire_dtype = jnp.bfloat16 if _WIRE == "bf16" else jnp.float32
    n_ocp = 5 + 2 * SUB

    def body(x_ref, w_ref, out_ref,
             xv_ref, wv_ref, ov_ref, xs_ref, cw_ref, ccw_ref,
             x_sems, w_sems, o_sems,
             cw_send, cw_recv, ccw_send, ccw_recv):
        my_pos = lax.axis_index("i")
        left = lax.rem(my_pos - 1 + N_DEV, N_DEV)
        right = lax.rem(my_pos + 1, N_DEV)

        def cw_sl(s):
            return pl.ds(s * m_sub, m_sub)

        def ccw_sl(s):
            return pl.ds(m_half + s * m_sub, m_sub)

        wcp = pltpu.make_async_copy(w_ref, wv_ref, w_sems.at[0])
        wcp.start()
        xcps = []
        for s in range(SUB):
            c1 = pltpu.make_async_copy(
                x_ref.at[cw_sl(s)], xv_ref.at[cw_sl(s)], x_sems.at[0, s])
            c2 = pltpu.make_async_copy(
                x_ref.at[ccw_sl(s)], xv_ref.at[ccw_sl(s)], x_sems.at[1, s])
            c1.start()
            c2.start()
            xcps.append((c1, c2))

        barrier_sem = pltpu.get_barrier_semaphore()
        for nbr in (left, right):
            pl.semaphore_signal(
                barrier_sem, inc=1,
                device_id=(nbr,), device_id_type=pl.DeviceIdType.MESH,
            )
        pl.semaphore_wait(barrier_sem, 2)

        hop0_src = xv_ref if wire_dtype == jnp.float32 else xs_ref

        def sub_rdma(h, s):
            sub_slice = pl.ds(s * m_sub, m_sub)
            if h == 0:
                cw_src = hop0_src.at[cw_sl(s)]
                ccw_src = hop0_src.at[ccw_sl(s)]
            else:
                cw_src = cw_ref.at[h - 1, sub_slice]
                ccw_src = ccw_ref.at[h - 1, sub_slice]
            cw = pltpu.make_async_remote_copy(
                src_ref=cw_src, dst_ref=cw_ref.at[h, sub_slice],
                send_sem=cw_send.at[h, s], recv_sem=cw_recv.at[h, s],
                device_id=(right,), device_id_type=pl.DeviceIdType.MESH,
            )
            ccw = pltpu.make_async_remote_copy(
                src_ref=ccw_src, dst_ref=ccw_ref.at[h, sub_slice],
                send_sem=ccw_send.at[h, s], recv_sem=ccw_recv.at[h, s],
                device_id=(left,), device_id_type=pl.DeviceIdType.MESH,
            )
            return cw, ccw

        ocps = []

        def gemm_store(buf, out_row, rows):
            ov_ref[pl.ds(out_row, rows), :] = jnp.maximum(
                jnp.dot(buf.astype(jnp.float32), wv_ref[...],
                        preferred_element_type=jnp.float32),
                0.0,
            )
            ocp = pltpu.make_async_copy(
                ov_ref.at[pl.ds(out_row, rows)],
                out_ref.at[pl.ds(out_row, rows)],
                o_sems.at[len(ocps)],
            )
            ocp.start()
            ocps.append(ocp)

        def hop_compute(h, sub=None):
            cw_origin = lax.rem(my_pos - h - 1 + 2 * N_DEV, N_DEV)
            ccw_origin = lax.rem(my_pos + h + 1, N_DEV)
            if sub is None:
                gemm_store(cw_ref[h], cw_origin * m_per, m_half)
                gemm_store(ccw_ref[h], ccw_origin * m_per + m_half, m_half)
            else:
                s, direction = sub
                if direction == "cw":
                    gemm_store(cw_ref[h, pl.ds(s * m_sub, m_sub)],
                               cw_origin * m_per + s * m_sub, m_sub)
                else:
                    gemm_store(ccw_ref[h, pl.ds(s * m_sub, m_sub)],
                               ccw_origin * m_per + m_half + s * m_sub, m_sub)

        do_compute = _MODE != "comm"
        started = []

        hop0 = []
        for s in range(SUB):
            c1, c2 = xcps[s]
            c1.wait()
            c2.wait()
            if wire_dtype != jnp.float32:
                xs_ref[cw_sl(s), :] = xv_ref[cw_sl(s), :].astype(wire_dtype)
                xs_ref[ccw_sl(s), :] = xv_ref[ccw_sl(s), :].astype(wire_dtype)
            cw, ccw = sub_rdma(0, s)
            cw.start()
            ccw.start()
            hop0.append((cw, ccw))
            started.append((cw, ccw))
        prev = hop0

        if do_compute:
            wcp.wait()
            gemm_store(xv_ref[...], my_pos * m_per, m_per)

        for h in range(1, N_HOP):
            cur = []
            for s in range(SUB):
                pcw, pccw = prev[s]
                cw, ccw = sub_rdma(h, s)
                pcw.wait_recv()
                cw.start()
                pccw.wait_recv()
                ccw.start()
                cur.append((cw, ccw))
                started.append((cw, ccw))
            prev = cur
            if do_compute:
                hop_compute(h - 1)

        for s in range(SUB):
            cw, ccw = prev[s]
            cw.wait_recv()
            if do_compute:
                hop_compute(N_HOP - 1, sub=(s, "cw"))
            ccw.wait_recv()
            if do_compute:
                hop_compute(N_HOP - 1, sub=(s, "ccw"))

        for ocp in ocps:
            ocp.wait()
        for cw, ccw in started:
            cw.wait_send()
            ccw.wait_send()

    return pl.pallas_call(
        body,
        out_shape=jax.ShapeDtypeStruct((N_DEV * m_per, n_per), jnp.float32),
        in_specs=[
            pl.BlockSpec(memory_space=pl.ANY),
            pl.BlockSpec(memory_space=pl.ANY),
        ],
        out_specs=pl.BlockSpec(memory_space=pl.ANY),
        scratch_shapes=[
            pltpu.VMEM((m_per, k), jnp.float32),
            pltpu.VMEM((k, n_per), jnp.float32),
            pltpu.VMEM((N_DEV * m_per, n_per), jnp.float32),
            pltpu.VMEM((m_per, k), wire_dtype),
            pltpu.VMEM((N_HOP, m_half, k), wire_dtype),
            pltpu.VMEM((N_HOP, m_half, k), wire_dtype),
            pltpu.SemaphoreType.DMA((2, SUB)),
            pltpu.SemaphoreType.DMA((1,)),
            pltpu.SemaphoreType.DMA((n_ocp,)),
            pltpu.SemaphoreType.DMA((N_HOP, SUB)),
            pltpu.SemaphoreType.DMA((N_HOP, SUB)),
            pltpu.SemaphoreType.DMA((N_HOP, SUB)),
            pltpu.SemaphoreType.DMA((N_HOP, SUB)),
        ],
        compiler_params=pltpu.CompilerParams(collective_id=0),
    )(x, w_mat)


# device time: 47239 ns/iter; 1.0049x vs baseline; 1.0049x over previous
import os

import jax
import jax.numpy as jnp
from jax import lax
from jax.experimental import pallas as pl
from jax.experimental.pallas import tpu as pltpu

N_DEV = 4
N_HOP = N_DEV - 1
SUB = int(os.environ.get("KSUB", "4"))
_MODE = os.environ.get("KMODE", "full")
_WIRE = os.environ.get("KWIRE", "bf16")


def kernel(x, w_mat):
    m_per, k = x.shape
    _, n_per = w_mat.shape
    m_half = m_per // 2
    m_sub = m_half // SUB
    wire_dtype = jnp.bfloat16 if _WIRE == "bf16" else jnp.float32
    n_ocp = 5 + 2 * SUB

    def body(x_ref, w_ref, out_ref,
             xv_ref, wv_ref, ov_ref, xs_ref, cw_ref, ccw_ref,
             x_sems, w_sems, o_sems,
             cw_send, cw_recv, ccw_send, ccw_recv):
        my_pos = lax.axis_index("i")
        left = lax.rem(my_pos - 1 + N_DEV, N_DEV)
        right = lax.rem(my_pos + 1, N_DEV)

        def cw_sl(s):
            return pl.ds(s * m_sub, m_sub)

        def ccw_sl(s):
            return pl.ds(m_half + s * m_sub, m_sub)

        wcp = pltpu.make_async_copy(w_ref, wv_ref, w_sems.at[0])
        xcps = []
        for s in range(SUB):
            c1 = pltpu.make_async_copy(
                x_ref.at[cw_sl(s)], xv_ref.at[cw_sl(s)], x_sems.at[0, s])
            c2 = pltpu.make_async_copy(
                x_ref.at[ccw_sl(s)], xv_ref.at[ccw_sl(s)], x_sems.at[1, s])
            c1.start()
            c2.start()
            xcps.append((c1, c2))

        barrier_sem = pltpu.get_barrier_semaphore()
        for nbr in (left, right):
            pl.semaphore_signal(
                barrier_sem, inc=1,
                device_id=(nbr,), device_id_type=pl.DeviceIdType.MESH,
            )
        pl.semaphore_wait(barrier_sem, 2)

        hop0_src = xv_ref if wire_dtype == jnp.float32 else xs_ref

        def sub_rdma(h, s):
            sub_slice = pl.ds(s * m_sub, m_sub)
            if h == 0:
                cw_src = hop0_src.at[cw_sl(s)]
                ccw_src = hop0_src.at[ccw_sl(s)]
            else:
                cw_src = cw_ref.at[h - 1, sub_slice]
                ccw_src = ccw_ref.at[h - 1, sub_slice]
            cw = pltpu.make_async_remote_copy(
                src_ref=cw_src, dst_ref=cw_ref.at[h, sub_slice],
                send_sem=cw_send.at[h, s], recv_sem=cw_recv.at[h, s],
                device_id=(right,), device_id_type=pl.DeviceIdType.MESH,
            )
            ccw = pltpu.make_async_remote_copy(
                src_ref=ccw_src, dst_ref=ccw_ref.at[h, sub_slice],
                send_sem=ccw_send.at[h, s], recv_sem=ccw_recv.at[h, s],
                device_id=(left,), device_id_type=pl.DeviceIdType.MESH,
            )
            return cw, ccw

        ocps = []

        def gemm_store(buf, out_row, rows):
            ov_ref[pl.ds(out_row, rows), :] = jnp.maximum(
                jnp.dot(buf.astype(jnp.float32), wv_ref[...],
                        preferred_element_type=jnp.float32),
                0.0,
            )
            ocp = pltpu.make_async_copy(
                ov_ref.at[pl.ds(out_row, rows)],
                out_ref.at[pl.ds(out_row, rows)],
                o_sems.at[len(ocps)],
            )
            ocp.start()
            ocps.append(ocp)

        def hop_compute(h, sub=None):
            cw_origin = lax.rem(my_pos - h - 1 + 2 * N_DEV, N_DEV)
            ccw_origin = lax.rem(my_pos + h + 1, N_DEV)
            if sub is None:
                gemm_store(cw_ref[h], cw_origin * m_per, m_half)
                gemm_store(ccw_ref[h], ccw_origin * m_per + m_half, m_half)
            else:
                s, direction = sub
                if direction == "cw":
                    gemm_store(cw_ref[h, pl.ds(s * m_sub, m_sub)],
                               cw_origin * m_per + s * m_sub, m_sub)
                else:
                    gemm_store(ccw_ref[h, pl.ds(s * m_sub, m_sub)],
                               ccw_origin * m_per + m_half + s * m_sub, m_sub)

        do_compute = _MODE != "comm"
        started = []

        hop0 = []
        for s in range(SUB):
            c1, c2 = xcps[s]
            c1.wait()
            c2.wait()
            if wire_dtype != jnp.float32:
                xs_ref[cw_sl(s), :] = xv_ref[cw_sl(s), :].astype(wire_dtype)
                xs_ref[ccw_sl(s), :] = xv_ref[ccw_sl(s), :].astype(wire_dtype)
            cw, ccw = sub_rdma(0, s)
            cw.start()
            ccw.start()
            hop0.append((cw, ccw))
            started.append((cw, ccw))
        prev = hop0

        wcp.start()

        if do_compute:
            wcp.wait()
            gemm_store(xv_ref[...], my_pos * m_per, m_per)

        for h in range(1, N_HOP):
            cur = []
            for s in range(SUB):
                pcw, pccw = prev[s]
                cw, ccw = sub_rdma(h, s)
                pcw.wait_recv()
                cw.start()
                pccw.wait_recv()
                ccw.start()
                cur.append((cw, ccw))
                started.append((cw, ccw))
            prev = cur
            if do_compute:
                hop_compute(h - 1)

        for s in range(SUB):
            cw, ccw = prev[s]
            cw.wait_recv()
            if do_compute:
                hop_compute(N_HOP - 1, sub=(s, "cw"))
            ccw.wait_recv()
            if do_compute:
                hop_compute(N_HOP - 1, sub=(s, "ccw"))

        for ocp in ocps:
            ocp.wait()
        for cw, ccw in started:
            cw.wait_send()
            ccw.wait_send()

    return pl.pallas_call(
        body,
        out_shape=jax.ShapeDtypeStruct((N_DEV * m_per, n_per), jnp.float32),
        in_specs=[
            pl.BlockSpec(memory_space=pl.ANY),
            pl.BlockSpec(memory_space=pl.ANY),
        ],
        out_specs=pl.BlockSpec(memory_space=pl.ANY),
        scratch_shapes=[
            pltpu.VMEM((m_per, k), jnp.float32),
            pltpu.VMEM((k, n_per), jnp.float32),
            pltpu.VMEM((N_DEV * m_per, n_per), jnp.float32),
            pltpu.VMEM((m_per, k), wire_dtype),
            pltpu.VMEM((N_HOP, m_half, k), wire_dtype),
            pltpu.VMEM((N_HOP, m_half, k), wire_dtype),
            pltpu.SemaphoreType.DMA((2, SUB)),
            pltpu.SemaphoreType.DMA((1,)),
            pltpu.SemaphoreType.DMA((n_ocp,)),
            pltpu.SemaphoreType.DMA((N_HOP, SUB)),
            pltpu.SemaphoreType.DMA((N_HOP, SUB)),
            pltpu.SemaphoreType.DMA((N_HOP, SUB)),
            pltpu.SemaphoreType.DMA((N_HOP, SUB)),
        ],
        compiler_params=pltpu.CompilerParams(collective_id=0),
    )(x, w_mat)


# device time: 42805 ns/iter; 1.1090x vs baseline; 1.1036x over previous
import os

import jax
import jax.numpy as jnp
from jax import lax
from jax.experimental import pallas as pl
from jax.experimental.pallas import tpu as pltpu

N_DEV = 4
N_HOP = N_DEV - 1
SUB = int(os.environ.get("KSUB", "4"))
_MODE = os.environ.get("KMODE", "full")
_WIRE = os.environ.get("KWIRE", "bf16")


def kernel(x, w_mat):
    x = pltpu.with_memory_space_constraint(x, pltpu.MemorySpace.HBM)
    w_mat = pltpu.with_memory_space_constraint(w_mat, pltpu.MemorySpace.HBM)
    m_per, k = x.shape
    _, n_per = w_mat.shape
    m_half = m_per // 2
    m_sub = m_half // SUB
    wire_dtype = jnp.bfloat16 if _WIRE == "bf16" else jnp.float32
    n_ocp = 5 + 2 * SUB

    def body(x_ref, w_ref, out_ref,
             xv_ref, wv_ref, ov_ref, xs_ref, cw_ref, ccw_ref,
             x_sems, w_sems, o_sems,
             cw_send, cw_recv, ccw_send, ccw_recv):
        my_pos = lax.axis_index("i")
        left = lax.rem(my_pos - 1 + N_DEV, N_DEV)
        right = lax.rem(my_pos + 1, N_DEV)

        def cw_sl(s):
            return pl.ds(s * m_sub, m_sub)

        def ccw_sl(s):
            return pl.ds(m_half + s * m_sub, m_sub)

        wcp = pltpu.make_async_copy(w_ref, wv_ref, w_sems.at[0])
        xcps = []
        for s in range(SUB):
            c1 = pltpu.make_async_copy(
                x_ref.at[cw_sl(s)], xv_ref.at[cw_sl(s)], x_sems.at[0, s])
            c2 = pltpu.make_async_copy(
                x_ref.at[ccw_sl(s)], xv_ref.at[ccw_sl(s)], x_sems.at[1, s])
            c1.start()
            c2.start()
            xcps.append((c1, c2))

        barrier_sem = pltpu.get_barrier_semaphore()
        for nbr in (left, right):
            pl.semaphore_signal(
                barrier_sem, inc=1,
                device_id=(nbr,), device_id_type=pl.DeviceIdType.MESH,
            )
        pl.semaphore_wait(barrier_sem, 2)

        hop0_src = xv_ref if wire_dtype == jnp.float32 else xs_ref

        def sub_rdma(h, s):
            sub_slice = pl.ds(s * m_sub, m_sub)
            if h == 0:
                cw_src = hop0_src.at[cw_sl(s)]
                ccw_src = hop0_src.at[ccw_sl(s)]
            else:
                cw_src = cw_ref.at[h - 1, sub_slice]
                ccw_src = ccw_ref.at[h - 1, sub_slice]
            cw = pltpu.make_async_remote_copy(
                src_ref=cw_src, dst_ref=cw_ref.at[h, sub_slice],
                send_sem=cw_send.at[h, s], recv_sem=cw_recv.at[h, s],
                device_id=(right,), device_id_type=pl.DeviceIdType.MESH,
            )
            ccw = pltpu.make_async_remote_copy(
                src_ref=ccw_src, dst_ref=ccw_ref.at[h, sub_slice],
                send_sem=ccw_send.at[h, s], recv_sem=ccw_recv.at[h, s],
                device_id=(left,), device_id_type=pl.DeviceIdType.MESH,
            )
            return cw, ccw

        ocps = []

        def gemm_store(buf, out_row, rows):
            ov_ref[pl.ds(out_row, rows), :] = jnp.maximum(
                jnp.dot(buf.astype(jnp.float32), wv_ref[...],
                        preferred_element_type=jnp.float32),
                0.0,
            )
            ocp = pltpu.make_async_copy(
                ov_ref.at[pl.ds(out_row, rows)],
                out_ref.at[pl.ds(out_row, rows)],
                o_sems.at[len(ocps)],
            )
            ocp.start()
            ocps.append(ocp)

        def hop_compute(h, sub=None):
            cw_origin = lax.rem(my_pos - h - 1 + 2 * N_DEV, N_DEV)
            ccw_origin = lax.rem(my_pos + h + 1, N_DEV)
            if sub is None:
                gemm_store(cw_ref[h], cw_origin * m_per, m_half)
                gemm_store(ccw_ref[h], ccw_origin * m_per + m_half, m_half)
            else:
                s, direction = sub
                if direction == "cw":
                    gemm_store(cw_ref[h, pl.ds(s * m_sub, m_sub)],
                               cw_origin * m_per + s * m_sub, m_sub)
                else:
                    gemm_store(ccw_ref[h, pl.ds(s * m_sub, m_sub)],
                               ccw_origin * m_per + m_half + s * m_sub, m_sub)

        do_compute = _MODE != "comm"
        started = []

        hop0 = []
        for s in range(SUB):
            c1, c2 = xcps[s]
            c1.wait()
            c2.wait()
            if wire_dtype != jnp.float32:
                xs_ref[cw_sl(s), :] = xv_ref[cw_sl(s), :].astype(wire_dtype)
                xs_ref[ccw_sl(s), :] = xv_ref[ccw_sl(s), :].astype(wire_dtype)
            cw, ccw = sub_rdma(0, s)
            cw.start()
            ccw.start()
            hop0.append((cw, ccw))
            started.append((cw, ccw))
        prev = hop0

        wcp.start()

        if do_compute:
            wcp.wait()
            gemm_store(xv_ref[...], my_pos * m_per, m_per)

        for h in range(1, N_HOP):
            cur = []
            for s in range(SUB):
                pcw, pccw = prev[s]
                cw, ccw = sub_rdma(h, s)
                pcw.wait_recv()
                cw.start()
                pccw.wait_recv()
                ccw.start()
                cur.append((cw, ccw))
                started.append((cw, ccw))
            prev = cur
            if do_compute:
                hop_compute(h - 1)

        for s in range(SUB):
            cw, ccw = prev[s]
            cw.wait_recv()
            if do_compute:
                hop_compute(N_HOP - 1, sub=(s, "cw"))
            ccw.wait_recv()
            if do_compute:
                hop_compute(N_HOP - 1, sub=(s, "ccw"))

        for ocp in ocps:
            ocp.wait()
        for cw, ccw in started:
            cw.wait_send()
            ccw.wait_send()

    return pl.pallas_call(
        body,
        out_shape=jax.ShapeDtypeStruct((N_DEV * m_per, n_per), jnp.float32),
        in_specs=[
            pl.BlockSpec(memory_space=pltpu.MemorySpace.HBM),
            pl.BlockSpec(memory_space=pltpu.MemorySpace.HBM),
        ],
        out_specs=pl.BlockSpec(memory_space=pltpu.MemorySpace.HBM),
        scratch_shapes=[
            pltpu.VMEM((m_per, k), jnp.float32),
            pltpu.VMEM((k, n_per), jnp.float32),
            pltpu.VMEM((N_DEV * m_per, n_per), jnp.float32),
            pltpu.VMEM((m_per, k), wire_dtype),
            pltpu.VMEM((N_HOP, m_half, k), wire_dtype),
            pltpu.VMEM((N_HOP, m_half, k), wire_dtype),
            pltpu.SemaphoreType.DMA((2, SUB)),
            pltpu.SemaphoreType.DMA((1,)),
            pltpu.SemaphoreType.DMA((n_ocp,)),
            pltpu.SemaphoreType.DMA((N_HOP, SUB)),
            pltpu.SemaphoreType.DMA((N_HOP, SUB)),
            pltpu.SemaphoreType.DMA((N_HOP, SUB)),
            pltpu.SemaphoreType.DMA((N_HOP, SUB)),
        ],
        compiler_params=pltpu.CompilerParams(collective_id=0),
    )(x, w_mat)
